# Initial kernel scaffold; baseline (speedup 1.0000x reference)
import jax
import jax.numpy as jnp
from jax.experimental import pallas as pl


def kernel(x, pos, edge_index, freq, W_x, W_rbf_emb, b_rbf_emb, W_emb, b_emb, W_rbf_out, W_out_lins, b_out_lins, W_final):
    raise NotImplementedError("write your pallas kernel here")


# speedup vs baseline: 2.8844x; 2.8844x over previous
import functools

import jax
import jax.numpy as jnp
from jax import lax
from jax.experimental import pallas as pl
from jax.experimental.pallas import tpu as pltpu
from jax.experimental.pallas import tpu_sc as plsc

N = 10000
E = 320000
H = 128
CUTOFF = 5.0
P_EXP = 5
NUM_OUT_LAYERS = 3

NC = 2
NS = 16
NW = NC * NS
CHUNK = 128
NCHUNK = E // CHUNK
CPW = -(-NCHUNK // NW)
NP = 10240
ROWS_PER_SUB = NP // NS
EP = 327680

BN = 2000
BE = 4096


def _swish(v):
    return v * (1.0 / (1.0 + jnp.exp(-v)))


def _prep_body(x_ref, wxt_ref, wit_ref, wjt_ref, ti_ref, tj_ref):
    h = jnp.dot(x_ref[...], wxt_ref[...], preferred_element_type=jnp.float32)
    ti_ref[...] = jnp.dot(h, wit_ref[...], preferred_element_type=jnp.float32)
    tj_ref[...] = jnp.dot(h, wjt_ref[...], preferred_element_type=jnp.float32)


def _prep(x, wxt, wit, wjt):
    grid = N // BN
    return pl.pallas_call(
        _prep_body,
        grid=(grid,),
        in_specs=[
            pl.BlockSpec((BN, H), lambda b: (b, 0)),
            pl.BlockSpec((H, H), lambda b: (0, 0)),
            pl.BlockSpec((H, H), lambda b: (0, 0)),
            pl.BlockSpec((H, H), lambda b: (0, 0)),
        ],
        out_specs=[
            pl.BlockSpec((BN, H), lambda b: (b, 0)),
            pl.BlockSpec((BN, H), lambda b: (b, 0)),
        ],
        out_shape=[
            jax.ShapeDtypeStruct((N, H), jnp.float32),
            jax.ShapeDtypeStruct((N, H), jnp.float32),
        ],
    )(x, wxt, wit, wjt)


def _gather_body(ti_hbm, tj_hbm, px_hbm, py_hbm, pz_hbm, ii_hbm, jj_hbm,
                 g_hbm, d2_hbm,
                 px_v, py_v, pz_v, ii_v, jj_v, gi_v, gj_v, d2_v, s0, s1):
    wid = lax.axis_index("s") * NC + lax.axis_index("c")
    pltpu.sync_copy(px_hbm, px_v)
    pltpu.sync_copy(py_hbm, py_v)
    pltpu.sync_copy(pz_hbm, pz_v)

    def chunk_body(k, carry):
        cid = wid + NW * k

        @pl.when(cid < NCHUNK)
        def _():
            base = pl.multiple_of(cid * CHUNK, CHUNK)
            pltpu.sync_copy(ii_hbm.at[pl.ds(base, CHUNK)], ii_v)
            pltpu.sync_copy(jj_hbm.at[pl.ds(base, CHUNK)], jj_v)
            a = pltpu.async_copy(ti_hbm.at[ii_v], gi_v, s0)
            b = pltpu.async_copy(tj_hbm.at[jj_v], gj_v, s1)
            for q in range(CHUNK // 16):
                sl = pl.ds(q * 16, 16)
                iq = ii_v[sl]
                jq = jj_v[sl]
                dx = plsc.load_gather(px_v, [iq]) - plsc.load_gather(px_v, [jq])
                dy = plsc.load_gather(py_v, [iq]) - plsc.load_gather(py_v, [jq])
                dz = plsc.load_gather(pz_v, [iq]) - plsc.load_gather(pz_v, [jq])
                d2_v[sl] = dx * dx + dy * dy + dz * dz
            a.wait()
            b.wait()

            def e_body(e, carry2):
                for v in range(H // 16):
                    sl = pl.ds(v * 16, 16)
                    gi_v[e, sl] = gi_v[e, sl] + gj_v[e, sl]
                return carry2

            lax.fori_loop(0, CHUNK, e_body, 0)
            pltpu.sync_copy(gi_v, g_hbm.at[pl.ds(base, CHUNK)])
            pltpu.sync_copy(d2_v, d2_hbm.at[pl.ds(base, CHUNK)])

        return carry

    lax.fori_loop(0, CPW, chunk_body, 0)


def _gather(ti, tj, px, py, pz, idx_i, idx_j):
    mesh = plsc.VectorSubcoreMesh(core_axis_name="c", subcore_axis_name="s", num_cores=NC, num_subcores=NS)
    f = pl.kernel(
        _gather_body,
        out_type=[
            jax.ShapeDtypeStruct((EP, H), jnp.float32),
            jax.ShapeDtypeStruct((EP,), jnp.float32),
        ],
        mesh=mesh,
        scratch_types=[
            pltpu.VMEM((N,), jnp.float32),
            pltpu.VMEM((N,), jnp.float32),
            pltpu.VMEM((N,), jnp.float32),
            pltpu.VMEM((CHUNK,), jnp.int32),
            pltpu.VMEM((CHUNK,), jnp.int32),
            pltpu.VMEM((CHUNK, H), jnp.float32),
            pltpu.VMEM((CHUNK, H), jnp.float32),
            pltpu.VMEM((CHUNK,), jnp.float32),
            pltpu.SemaphoreType.DMA,
            pltpu.SemaphoreType.DMA,
        ],
        compiler_params=pltpu.CompilerParams(needs_layout_passes=False),
    )
    return f(ti, tj, px, py, pz, idx_i, idx_j)


GR = BE // H


def _edge_body(g_ref, d2r_ref, freq_ref, wrbfe_ref, brbfe_ref, wrt_ref,
               bemb_ref, wrbfo_ref, t_ref):
    d2b = d2r_ref[...]
    e_id = lax.broadcasted_iota(jnp.int32, (BE, GR), 0)
    r_id = lax.broadcasted_iota(jnp.int32, (BE, GR), 1)
    onehot = (e_id // H == r_id).astype(jnp.float32)
    m_bcast = jnp.dot(onehot, d2b, preferred_element_type=jnp.float32)
    lane_e = lax.broadcasted_iota(jnp.int32, (BE, H), 0) % H
    lane_c = lax.broadcasted_iota(jnp.int32, (BE, H), 1)
    sel = (lane_e == lane_c).astype(jnp.float32)
    d2 = jnp.sum(m_bcast * sel, axis=1, keepdims=True)
    dist = jnp.sqrt(d2)
    u = dist * (1.0 / CUTOFF)
    p = P_EXP
    a = -(p + 1) * (p + 2) / 2.0
    b = p * (p + 2)
    c = -p * (p + 1) / 2.0
    u2 = u * u
    u4 = u2 * u2
    u5 = u4 * u
    u6 = u5 * u
    u7 = u6 * u
    env = 1.0 / u + a * u5 + b * u6 + c * u7
    rbf = env * jnp.sin(u * freq_ref[...])
    rbf_h = _swish(
        jnp.dot(rbf, wrbfe_ref[...], preferred_element_type=jnp.float32)
        + brbfe_ref[...])
    rproj = jnp.dot(rbf_h, wrt_ref[...], preferred_element_type=jnp.float32)
    m = _swish(g_ref[...] + rproj + bemb_ref[...])
    t_ref[...] = jnp.dot(
        rbf, wrbfo_ref[...], preferred_element_type=jnp.float32) * m


def _edge(g, d2r, freq_p, wrbfe_p, brbfe, wrt, bemb, wrbfo_p):
    grid = EP // BE
    return pl.pallas_call(
        _edge_body,
        grid=(grid,),
        in_specs=[
            pl.BlockSpec((BE, H), lambda b: (b, 0)),
            pl.BlockSpec((GR, H), lambda b: (b, 0)),
            pl.BlockSpec((1, 8), lambda b: (0, 0)),
            pl.BlockSpec((8, H), lambda b: (0, 0)),
            pl.BlockSpec((1, H), lambda b: (0, 0)),
            pl.BlockSpec((H, H), lambda b: (0, 0)),
            pl.BlockSpec((1, H), lambda b: (0, 0)),
            pl.BlockSpec((8, H), lambda b: (0, 0)),
        ],
        out_specs=pl.BlockSpec((BE, H), lambda b: (b, 0)),
        out_shape=jax.ShapeDtypeStruct((EP, H), jnp.float32),
    )(g, d2r, freq_p, wrbfe_p, brbfe, wrt, bemb, wrbfo_p)


def _scatter_body(t_hbm, ii_hbm, zeros_hbm, part_hbm, idx_v, t_v, acc_sh, sem):
    cc = lax.axis_index("c")
    ss = lax.axis_index("s")
    wid = ss * NC + cc

    row0 = pl.multiple_of(ss * ROWS_PER_SUB, ROWS_PER_SUB)
    pltpu.sync_copy(zeros_hbm.at[pl.ds(row0, ROWS_PER_SUB)],
                    acc_sh.at[pl.ds(row0, ROWS_PER_SUB)])
    plsc.subcore_barrier()

    def chunk_body(k, carry):
        cid = wid + NW * k

        @pl.when(cid < NCHUNK)
        def _():
            base = pl.multiple_of(cid * CHUNK, CHUNK)
            pltpu.sync_copy(ii_hbm.at[pl.ds(base, CHUNK)], idx_v)
            pltpu.async_copy(t_hbm.at[pl.ds(base, CHUNK)], t_v, sem).wait()
            pltpu.sync_copy(t_v, acc_sh.at[idx_v], add=True)

        return carry

    lax.fori_loop(0, CPW, chunk_body, 0)
    plsc.subcore_barrier()
    pltpu.sync_copy(acc_sh.at[pl.ds(row0, ROWS_PER_SUB)],
                    part_hbm.at[cc, pl.ds(row0, ROWS_PER_SUB)])


def _scatter(t, idx_i, zeros_nh):
    mesh = plsc.VectorSubcoreMesh(core_axis_name="c", subcore_axis_name="s", num_cores=NC, num_subcores=NS)
    f = pl.kernel(
        _scatter_body,
        out_type=jax.ShapeDtypeStruct((NC, NP, H), jnp.float32),
        mesh=mesh,
        scratch_types=[
            pltpu.VMEM((CHUNK,), jnp.int32),
            pltpu.VMEM((CHUNK, H), jnp.float32),
            pltpu.VMEM_SHARED((NP, H), jnp.float32),
            pltpu.SemaphoreType.DMA,
        ],
        compiler_params=pltpu.CompilerParams(needs_layout_passes=False),
    )
    return f(t, idx_i, zeros_nh)


def _mlp_body(p0_ref, p1_ref, w_ref, b_ref, wf_ref, out_ref):
    u = p0_ref[...] + p1_ref[...]
    for k in range(NUM_OUT_LAYERS):
        u = _swish(
            jnp.dot(u, w_ref[k], preferred_element_type=jnp.float32)
            + b_ref[pl.ds(k, 1)])
    out_ref[...] = jnp.dot(u, wf_ref[...], preferred_element_type=jnp.float32)


def _outmlp(p0, p1, w_stack_t, b_stack, wft):
    grid = N // BN
    return pl.pallas_call(
        _mlp_body,
        grid=(grid,),
        in_specs=[
            pl.BlockSpec((BN, H), lambda b: (b, 0)),
            pl.BlockSpec((BN, H), lambda b: (b, 0)),
            pl.BlockSpec((NUM_OUT_LAYERS, H, H), lambda b: (0, 0, 0)),
            pl.BlockSpec((NUM_OUT_LAYERS, H), lambda b: (0, 0)),
            pl.BlockSpec((H, H), lambda b: (0, 0)),
        ],
        out_specs=pl.BlockSpec((BN, H), lambda b: (b, 0)),
        out_shape=jax.ShapeDtypeStruct((N, H), jnp.float32),
    )(p0, p1, w_stack_t, b_stack, wft)


def kernel(x, pos, edge_index, freq, W_x, W_rbf_emb, b_rbf_emb, W_emb, b_emb,
           W_rbf_out, W_out_lins, b_out_lins, W_final):
    Wi = W_emb[:, :H]
    Wj = W_emb[:, H:2 * H]
    Wr = W_emb[:, 2 * H:]

    ti, tj = _prep(x, W_x.T, Wi.T, Wj.T)

    px = pos[:, 0]
    py = pos[:, 1]
    pz = pos[:, 2]
    idx_j = edge_index[0]
    idx_i = edge_index[1]

    g, d2 = _gather(ti, tj, px, py, pz, idx_i, idx_j)
    d2r = d2.reshape(EP // H, H)

    freq_p = jnp.pad(freq, (0, 2)).reshape(1, 8)
    wrbfe_p = jnp.pad(W_rbf_emb.T, ((0, 2), (0, 0)))
    wrbfo_p = jnp.pad(W_rbf_out.T, ((0, 2), (0, 0)))
    t = _edge(g, d2r, freq_p, wrbfe_p, b_rbf_emb.reshape(1, H), Wr.T,
              b_emb.reshape(1, H), wrbfo_p)

    parts = _scatter(t, idx_i, jnp.zeros((NP, H), jnp.float32))

    w_stack_t = jnp.transpose(W_out_lins, (0, 2, 1))
    P = _outmlp(parts[0], parts[1], w_stack_t, b_out_lins, W_final.T)
    return P


# speedup vs baseline: 3.9466x; 1.3683x over previous
import functools

import jax
import jax.numpy as jnp
from jax import lax
from jax.experimental import pallas as pl
from jax.experimental.pallas import tpu as pltpu
from jax.experimental.pallas import tpu_sc as plsc

N = 10000
E = 320000
H = 128
CUTOFF = 5.0
P_EXP = 5
NUM_OUT_LAYERS = 3

NC = 2
NS = 16
NW = NC * NS
CHUNK = 128
NCHUNK = E // CHUNK
CPW = -(-NCHUNK // NW)
NP = 10240
ROWS_PER_SUB = NP // NS
EP = 327680

BN = 2000
BE = 4096


def _swish(v):
    return v * (1.0 / (1.0 + jnp.exp(-v)))


def _prep_body(x_ref, wxt_ref, wit_ref, wjt_ref, ti_ref, tj_ref):
    h = jnp.dot(x_ref[...], wxt_ref[...], preferred_element_type=jnp.float32)
    ti_ref[...] = jnp.dot(h, wit_ref[...], preferred_element_type=jnp.float32)
    tj_ref[...] = jnp.dot(h, wjt_ref[...], preferred_element_type=jnp.float32)


def _prep(x, wxt, wit, wjt):
    grid = N // BN
    return pl.pallas_call(
        _prep_body,
        grid=(grid,),
        in_specs=[
            pl.BlockSpec((BN, H), lambda b: (b, 0)),
            pl.BlockSpec((H, H), lambda b: (0, 0)),
            pl.BlockSpec((H, H), lambda b: (0, 0)),
            pl.BlockSpec((H, H), lambda b: (0, 0)),
        ],
        out_specs=[
            pl.BlockSpec((BN, H), lambda b: (b, 0)),
            pl.BlockSpec((BN, H), lambda b: (b, 0)),
        ],
        out_shape=[
            jax.ShapeDtypeStruct((N, H), jnp.float32),
            jax.ShapeDtypeStruct((N, H), jnp.float32),
        ],
    )(x, wxt, wit, wjt)


def _gather_body(ti_hbm, tj_hbm, px_hbm, py_hbm, pz_hbm, ii_hbm, jj_hbm,
                 g_hbm, d2_hbm,
                 px_v, py_v, pz_v, ii_v, jj_v, gi_v, gj_v, d2_v, s0, s1):
    wid = lax.axis_index("s") * NC + lax.axis_index("c")
    pltpu.sync_copy(px_hbm, px_v)
    pltpu.sync_copy(py_hbm, py_v)
    pltpu.sync_copy(pz_hbm, pz_v)

    def chunk_body(k, carry):
        cid = wid + NW * k

        @pl.when(cid < NCHUNK)
        def _():
            base = pl.multiple_of(cid * CHUNK, CHUNK)
            pltpu.sync_copy(ii_hbm.at[pl.ds(base, CHUNK)], ii_v)
            pltpu.sync_copy(jj_hbm.at[pl.ds(base, CHUNK)], jj_v)
            a = pltpu.async_copy(ti_hbm.at[ii_v], gi_v, s0)
            b = pltpu.async_copy(tj_hbm.at[jj_v], gj_v, s1)
            for q in range(CHUNK // 16):
                sl = pl.ds(q * 16, 16)
                iq = ii_v[sl]
                jq = jj_v[sl]
                dx = plsc.load_gather(px_v, [iq]) - plsc.load_gather(px_v, [jq])
                dy = plsc.load_gather(py_v, [iq]) - plsc.load_gather(py_v, [jq])
                dz = plsc.load_gather(pz_v, [iq]) - plsc.load_gather(pz_v, [jq])
                d2_v[sl] = dx * dx + dy * dy + dz * dz
            a.wait()
            b.wait()

            def e_body(e, carry2):
                for v in range(H // 16):
                    sl = pl.ds(v * 16, 16)
                    gi_v[e, sl] = gi_v[e, sl] + gj_v[e, sl]
                return carry2

            lax.fori_loop(0, CHUNK, e_body, 0)
            pltpu.sync_copy(gi_v, g_hbm.at[pl.ds(base, CHUNK)])
            pltpu.sync_copy(d2_v, d2_hbm.at[pl.ds(base, CHUNK)])

        return carry

    lax.fori_loop(0, CPW, chunk_body, 0)


def _gather(ti, tj, px, py, pz, idx_i, idx_j):
    mesh = plsc.VectorSubcoreMesh(core_axis_name="c", subcore_axis_name="s", num_cores=NC, num_subcores=NS)
    f = pl.kernel(
        _gather_body,
        out_type=[
            jax.ShapeDtypeStruct((EP, H), jnp.float32),
            jax.ShapeDtypeStruct((EP,), jnp.float32),
        ],
        mesh=mesh,
        scratch_types=[
            pltpu.VMEM((N,), jnp.float32),
            pltpu.VMEM((N,), jnp.float32),
            pltpu.VMEM((N,), jnp.float32),
            pltpu.VMEM((CHUNK,), jnp.int32),
            pltpu.VMEM((CHUNK,), jnp.int32),
            pltpu.VMEM((CHUNK, H), jnp.float32),
            pltpu.VMEM((CHUNK, H), jnp.float32),
            pltpu.VMEM((CHUNK,), jnp.float32),
            pltpu.SemaphoreType.DMA,
            pltpu.SemaphoreType.DMA,
        ],
        compiler_params=pltpu.CompilerParams(needs_layout_passes=False),
    )
    return f(ti, tj, px, py, pz, idx_i, idx_j)


GR = BE // H


def _edge_body(g_ref, d2r_ref, freq_ref, onehot_ref, sel_ref, wrbfe_ref,
               brbfe_ref, wrt_ref, bemb_ref, wrbfo_ref, t_ref):
    d2b = d2r_ref[...]
    up = jnp.sqrt(d2b) * (1.0 / CUTOFF)
    p = P_EXP
    a = -(p + 1) * (p + 2) / 2.0
    b = p * (p + 2)
    c = -p * (p + 1) / 2.0
    u2 = up * up
    u4 = u2 * u2
    u5 = u4 * up
    u6 = u5 * up
    u7 = u6 * up
    envp = 1.0 / up + a * u5 + b * u6 + c * u7
    envs = jnp.where(d2b > 0, envp, 0.0)
    comps = [envs * jnp.sin(up * freq_ref[0:1, k:k + 1]) for k in range(6)]

    onehot = onehot_ref[...]
    sel = sel_ref[...]

    def unpack(packed):
        m_bcast = jnp.dot(onehot, packed, preferred_element_type=jnp.float32)
        return jnp.sum(m_bcast * sel, axis=1, keepdims=True)

    d2 = unpack(d2b)
    nanmask = jnp.where(d2 == 0, jnp.float32(jnp.nan), jnp.float32(1.0))
    cols = [unpack(cmp) for cmp in comps]
    cols.append(jnp.zeros((BE, 2), jnp.float32))
    rbf = jnp.concatenate(cols, axis=1)
    rbf_h = _swish(
        jnp.dot(rbf, wrbfe_ref[...], preferred_element_type=jnp.float32)
        + brbfe_ref[...])
    rproj = jnp.dot(rbf_h, wrt_ref[...], preferred_element_type=jnp.float32)
    m = _swish(g_ref[...] + rproj + bemb_ref[...])
    t_ref[...] = jnp.dot(
        rbf, wrbfo_ref[...], preferred_element_type=jnp.float32) * m * nanmask


def _edge(g, d2r, freq_p, onehot, sel, wrbfe_p, brbfe, wrt, bemb, wrbfo_p):
    grid = EP // BE
    return pl.pallas_call(
        _edge_body,
        grid=(grid,),
        in_specs=[
            pl.BlockSpec((BE, H), lambda b: (b, 0)),
            pl.BlockSpec((GR, H), lambda b: (b, 0)),
            pl.BlockSpec((1, 8), lambda b: (0, 0)),
            pl.BlockSpec((BE, GR), lambda b: (0, 0)),
            pl.BlockSpec((BE, H), lambda b: (0, 0)),
            pl.BlockSpec((8, H), lambda b: (0, 0)),
            pl.BlockSpec((1, H), lambda b: (0, 0)),
            pl.BlockSpec((H, H), lambda b: (0, 0)),
            pl.BlockSpec((1, H), lambda b: (0, 0)),
            pl.BlockSpec((8, H), lambda b: (0, 0)),
        ],
        out_specs=pl.BlockSpec((BE, H), lambda b: (b, 0)),
        out_shape=jax.ShapeDtypeStruct((EP, H), jnp.float32),
    )(g, d2r, freq_p, onehot, sel, wrbfe_p, brbfe, wrt, bemb, wrbfo_p)


def _scatter_body(t_hbm, ii_hbm, zeros_hbm, part_hbm, idx_v, t_v, acc_sh, sem):
    cc = lax.axis_index("c")
    ss = lax.axis_index("s")
    wid = ss * NC + cc

    row0 = pl.multiple_of(ss * ROWS_PER_SUB, ROWS_PER_SUB)
    pltpu.sync_copy(zeros_hbm.at[pl.ds(row0, ROWS_PER_SUB)],
                    acc_sh.at[pl.ds(row0, ROWS_PER_SUB)])
    plsc.subcore_barrier()

    def chunk_body(k, carry):
        cid = wid + NW * k

        @pl.when(cid < NCHUNK)
        def _():
            base = pl.multiple_of(cid * CHUNK, CHUNK)
            pltpu.sync_copy(ii_hbm.at[pl.ds(base, CHUNK)], idx_v)
            pltpu.async_copy(t_hbm.at[pl.ds(base, CHUNK)], t_v, sem).wait()
            pltpu.sync_copy(t_v, acc_sh.at[idx_v], add=True)

        return carry

    lax.fori_loop(0, CPW, chunk_body, 0)
    plsc.subcore_barrier()
    pltpu.sync_copy(acc_sh.at[pl.ds(row0, ROWS_PER_SUB)],
                    part_hbm.at[cc, pl.ds(row0, ROWS_PER_SUB)])


def _scatter(t, idx_i, zeros_nh):
    mesh = plsc.VectorSubcoreMesh(core_axis_name="c", subcore_axis_name="s", num_cores=NC, num_subcores=NS)
    f = pl.kernel(
        _scatter_body,
        out_type=jax.ShapeDtypeStruct((NC, NP, H), jnp.float32),
        mesh=mesh,
        scratch_types=[
            pltpu.VMEM((CHUNK,), jnp.int32),
            pltpu.VMEM((CHUNK, H), jnp.float32),
            pltpu.VMEM_SHARED((NP, H), jnp.float32),
            pltpu.SemaphoreType.DMA,
        ],
        compiler_params=pltpu.CompilerParams(needs_layout_passes=False),
    )
    return f(t, idx_i, zeros_nh)


def _mlp_body(p0_ref, p1_ref, w_ref, b_ref, wf_ref, out_ref):
    u = p0_ref[...] + p1_ref[...]
    for k in range(NUM_OUT_LAYERS):
        u = _swish(
            jnp.dot(u, w_ref[k], preferred_element_type=jnp.float32)
            + b_ref[pl.ds(k, 1)])
    out_ref[...] = jnp.dot(u, wf_ref[...], preferred_element_type=jnp.float32)


def _outmlp(p0, p1, w_stack_t, b_stack, wft):
    grid = N // BN
    return pl.pallas_call(
        _mlp_body,
        grid=(grid,),
        in_specs=[
            pl.BlockSpec((BN, H), lambda b: (b, 0)),
            pl.BlockSpec((BN, H), lambda b: (b, 0)),
            pl.BlockSpec((NUM_OUT_LAYERS, H, H), lambda b: (0, 0, 0)),
            pl.BlockSpec((NUM_OUT_LAYERS, H), lambda b: (0, 0)),
            pl.BlockSpec((H, H), lambda b: (0, 0)),
        ],
        out_specs=pl.BlockSpec((BN, H), lambda b: (b, 0)),
        out_shape=jax.ShapeDtypeStruct((N, H), jnp.float32),
    )(p0, p1, w_stack_t, b_stack, wft)


def kernel(x, pos, edge_index, freq, W_x, W_rbf_emb, b_rbf_emb, W_emb, b_emb,
           W_rbf_out, W_out_lins, b_out_lins, W_final):
    Wi = W_emb[:, :H]
    Wj = W_emb[:, H:2 * H]
    Wr = W_emb[:, 2 * H:]

    ti, tj = _prep(x, W_x.T, Wi.T, Wj.T)

    px = pos[:, 0]
    py = pos[:, 1]
    pz = pos[:, 2]
    idx_j = edge_index[0]
    idx_i = edge_index[1]

    g, d2 = _gather(ti, tj, px, py, pz, idx_i, idx_j)
    d2r = d2.reshape(EP // H, H)

    freq_p = jnp.pad(freq, (0, 2)).reshape(1, 8)
    wrbfe_p = jnp.pad(W_rbf_emb.T, ((0, 2), (0, 0)))
    wrbfo_p = jnp.pad(W_rbf_out.T, ((0, 2), (0, 0)))
    e_ar = jnp.arange(BE, dtype=jnp.int32)
    onehot = (e_ar[:, None] // H == jnp.arange(GR, dtype=jnp.int32)[None, :]
              ).astype(jnp.float32)
    sel = (e_ar[:, None] % H == jnp.arange(H, dtype=jnp.int32)[None, :]
           ).astype(jnp.float32)
    t = _edge(g, d2r, freq_p, onehot, sel, wrbfe_p, b_rbf_emb.reshape(1, H),
              Wr.T, b_emb.reshape(1, H), wrbfo_p)

    parts = _scatter(t, idx_i, jnp.zeros((NP, H), jnp.float32))

    w_stack_t = jnp.transpose(W_out_lins, (0, 2, 1))
    P = _outmlp(parts[0], parts[1], w_stack_t, b_out_lins, W_final.T)
    return P


# speedup vs baseline: 5.2422x; 1.3283x over previous
import functools

import jax
import jax.numpy as jnp
from jax import lax
from jax.experimental import pallas as pl
from jax.experimental.pallas import tpu as pltpu
from jax.experimental.pallas import tpu_sc as plsc

N = 10000
E = 320000
H = 128
CUTOFF = 5.0
P_EXP = 5
NUM_OUT_LAYERS = 3

NC = 2
NS = 16
NW = NC * NS
CHUNK = 128
SL = 4
ES = E // SL
NCHUNK = ES // CHUNK
CPW = -(-NCHUNK // NW)
NP = 10240
ROWS_PER_SUB = NP // NS
EPS = 81920

BN = 2000
BE = 4096


def _swish(v):
    return v * (1.0 / (1.0 + jnp.exp(-v)))


def _prep_body(x_ref, wxt_ref, wit_ref, wjt_ref, ti_ref, tj_ref):
    h = jnp.dot(x_ref[...], wxt_ref[...], preferred_element_type=jnp.float32)
    ti_ref[...] = jnp.dot(h, wit_ref[...], preferred_element_type=jnp.float32)
    tj_ref[...] = jnp.dot(h, wjt_ref[...], preferred_element_type=jnp.float32)


def _prep(x, wxt, wit, wjt):
    grid = N // BN
    return pl.pallas_call(
        _prep_body,
        grid=(grid,),
        in_specs=[
            pl.BlockSpec((BN, H), lambda b: (b, 0)),
            pl.BlockSpec((H, H), lambda b: (0, 0)),
            pl.BlockSpec((H, H), lambda b: (0, 0)),
            pl.BlockSpec((H, H), lambda b: (0, 0)),
        ],
        out_specs=[
            pl.BlockSpec((BN, H), lambda b: (b, 0)),
            pl.BlockSpec((BN, H), lambda b: (b, 0)),
        ],
        out_shape=[
            jax.ShapeDtypeStruct((N, H), jnp.float32),
            jax.ShapeDtypeStruct((N, H), jnp.float32),
        ],
    )(x, wxt, wit, wjt)


def _gather_body(ti_hbm, tj_hbm, px_hbm, py_hbm, pz_hbm, ii_hbm, jj_hbm,
                 g_hbm, d2_hbm,
                 px_v, py_v, pz_v, ii_v, jj_v, gi_v, gj_v, d2_v, s0, s1):
    wid = lax.axis_index("s") * NC + lax.axis_index("c")
    pltpu.sync_copy(px_hbm, px_v)
    pltpu.sync_copy(py_hbm, py_v)
    pltpu.sync_copy(pz_hbm, pz_v)

    def chunk_body(k, carry):
        cid = wid + NW * k

        @pl.when(cid < NCHUNK)
        def _():
            base = pl.multiple_of(cid * CHUNK, CHUNK)
            pltpu.sync_copy(ii_hbm.at[pl.ds(base, CHUNK)], ii_v)
            pltpu.sync_copy(jj_hbm.at[pl.ds(base, CHUNK)], jj_v)
            a = pltpu.async_copy(ti_hbm.at[ii_v], gi_v, s0)
            b = pltpu.async_copy(tj_hbm.at[jj_v], gj_v, s1)
            for q in range(CHUNK // 16):
                sl = pl.ds(q * 16, 16)
                iq = ii_v[sl]
                jq = jj_v[sl]
                dx = plsc.load_gather(px_v, [iq]) - plsc.load_gather(px_v, [jq])
                dy = plsc.load_gather(py_v, [iq]) - plsc.load_gather(py_v, [jq])
                dz = plsc.load_gather(pz_v, [iq]) - plsc.load_gather(pz_v, [jq])
                d2_v[sl] = dx * dx + dy * dy + dz * dz
            a.wait()
            b.wait()

            def e_body(e, carry2):
                for v in range(H // 16):
                    sl = pl.ds(v * 16, 16)
                    gi_v[e, sl] = gi_v[e, sl] + gj_v[e, sl]
                return carry2

            lax.fori_loop(0, CHUNK, e_body, 0)
            pltpu.sync_copy(gi_v, g_hbm.at[pl.ds(base, CHUNK)])
            pltpu.sync_copy(d2_v, d2_hbm.at[pl.ds(base, CHUNK)])

        return carry

    lax.fori_loop(0, CPW, chunk_body, 0)


def _gather(ti, tj, px, py, pz, idx_i, idx_j):
    mesh = plsc.VectorSubcoreMesh(core_axis_name="c", subcore_axis_name="s", num_cores=NC, num_subcores=NS)
    f = pl.kernel(
        _gather_body,
        out_type=[
            jax.ShapeDtypeStruct((EPS, H), jnp.float32),
            jax.ShapeDtypeStruct((EPS,), jnp.float32),
        ],
        mesh=mesh,
        scratch_types=[
            pltpu.VMEM((N,), jnp.float32),
            pltpu.VMEM((N,), jnp.float32),
            pltpu.VMEM((N,), jnp.float32),
            pltpu.VMEM((CHUNK,), jnp.int32),
            pltpu.VMEM((CHUNK,), jnp.int32),
            pltpu.VMEM((CHUNK, H), jnp.float32),
            pltpu.VMEM((CHUNK, H), jnp.float32),
            pltpu.VMEM((CHUNK,), jnp.float32),
            pltpu.SemaphoreType.DMA,
            pltpu.SemaphoreType.DMA,
        ],
        compiler_params=pltpu.CompilerParams(needs_layout_passes=False),
    )
    return f(ti, tj, px, py, pz, idx_i, idx_j)


GR = BE // H


def _edge_body(g_ref, d2r_ref, freq_ref, onehot_ref, sel_ref, wrbfe_ref,
               brbfe_ref, wrt_ref, bemb_ref, wrbfo_ref, t_ref):
    d2b = d2r_ref[...]
    up = jnp.sqrt(d2b) * (1.0 / CUTOFF)
    p = P_EXP
    a = -(p + 1) * (p + 2) / 2.0
    b = p * (p + 2)
    c = -p * (p + 1) / 2.0
    u2 = up * up
    u4 = u2 * u2
    u5 = u4 * up
    u6 = u5 * up
    u7 = u6 * up
    envp = 1.0 / up + a * u5 + b * u6 + c * u7
    envs = jnp.where(d2b > 0, envp, 0.0)
    comps = [envs * jnp.sin(up * freq_ref[0:1, k:k + 1]) for k in range(6)]

    onehot = onehot_ref[...]
    sel = sel_ref[...]

    def unpack(packed):
        m_bcast = jnp.dot(onehot, packed, preferred_element_type=jnp.float32)
        return jnp.sum(m_bcast * sel, axis=1, keepdims=True)

    d2 = unpack(d2b)
    nanmask = jnp.where(d2 == 0, jnp.float32(jnp.nan), jnp.float32(1.0))
    cols = [unpack(cmp) for cmp in comps]
    cols.append(jnp.zeros((BE, 2), jnp.float32))
    rbf = jnp.concatenate(cols, axis=1)
    rbf_h = _swish(
        jnp.dot(rbf, wrbfe_ref[...], preferred_element_type=jnp.float32)
        + brbfe_ref[...])
    rproj = jnp.dot(rbf_h, wrt_ref[...], preferred_element_type=jnp.float32)
    m = _swish(g_ref[...] + rproj + bemb_ref[...])
    t_ref[...] = jnp.dot(
        rbf, wrbfo_ref[...], preferred_element_type=jnp.float32) * m * nanmask


def _edge(g, d2r, freq_p, onehot, sel, wrbfe_p, brbfe, wrt, bemb, wrbfo_p):
    grid = EPS // BE
    return pl.pallas_call(
        _edge_body,
        grid=(grid,),
        in_specs=[
            pl.BlockSpec((BE, H), lambda b: (b, 0)),
            pl.BlockSpec((GR, H), lambda b: (b, 0)),
            pl.BlockSpec((1, 8), lambda b: (0, 0)),
            pl.BlockSpec((BE, GR), lambda b: (0, 0)),
            pl.BlockSpec((BE, H), lambda b: (0, 0)),
            pl.BlockSpec((8, H), lambda b: (0, 0)),
            pl.BlockSpec((1, H), lambda b: (0, 0)),
            pl.BlockSpec((H, H), lambda b: (0, 0)),
            pl.BlockSpec((1, H), lambda b: (0, 0)),
            pl.BlockSpec((8, H), lambda b: (0, 0)),
        ],
        out_specs=pl.BlockSpec((BE, H), lambda b: (b, 0)),
        out_shape=jax.ShapeDtypeStruct((EPS, H), jnp.float32),
    )(g, d2r, freq_p, onehot, sel, wrbfe_p, brbfe, wrt, bemb, wrbfo_p)


def _scatter_body(t_hbm, ii_hbm, zeros_hbm, part_hbm, idx_v, t_v, acc_sh, sem):
    cc = lax.axis_index("c")
    ss = lax.axis_index("s")
    wid = ss * NC + cc

    row0 = pl.multiple_of(ss * ROWS_PER_SUB, ROWS_PER_SUB)
    pltpu.sync_copy(zeros_hbm.at[pl.ds(row0, ROWS_PER_SUB)],
                    acc_sh.at[pl.ds(row0, ROWS_PER_SUB)])
    plsc.subcore_barrier()

    def chunk_body(k, carry):
        cid = wid + NW * k

        @pl.when(cid < NCHUNK)
        def _():
            base = pl.multiple_of(cid * CHUNK, CHUNK)
            pltpu.sync_copy(ii_hbm.at[pl.ds(base, CHUNK)], idx_v)
            pltpu.async_copy(t_hbm.at[pl.ds(base, CHUNK)], t_v, sem).wait()
            pltpu.sync_copy(t_v, acc_sh.at[idx_v], add=True)

        return carry

    lax.fori_loop(0, CPW, chunk_body, 0)
    plsc.subcore_barrier()
    pltpu.sync_copy(acc_sh.at[pl.ds(row0, ROWS_PER_SUB)],
                    part_hbm.at[cc, pl.ds(row0, ROWS_PER_SUB)])


def _scatter(t, idx_i, zeros_nh):
    mesh = plsc.VectorSubcoreMesh(core_axis_name="c", subcore_axis_name="s", num_cores=NC, num_subcores=NS)
    f = pl.kernel(
        _scatter_body,
        out_type=jax.ShapeDtypeStruct((NC, NP, H), jnp.float32),
        mesh=mesh,
        scratch_types=[
            pltpu.VMEM((CHUNK,), jnp.int32),
            pltpu.VMEM((CHUNK, H), jnp.float32),
            pltpu.VMEM_SHARED((NP, H), jnp.float32),
            pltpu.SemaphoreType.DMA,
        ],
        compiler_params=pltpu.CompilerParams(needs_layout_passes=False),
    )
    return f(t, idx_i, zeros_nh)


def _mlp_body(*refs):
    part_refs = refs[:2 * SL]
    w_ref, b_ref, wf_ref, out_ref = refs[2 * SL:]
    u = part_refs[0][...]
    for pr in part_refs[1:]:
        u = u + pr[...]
    for k in range(NUM_OUT_LAYERS):
        u = _swish(
            jnp.dot(u, w_ref[k], preferred_element_type=jnp.float32)
            + b_ref[pl.ds(k, 1)])
    out_ref[...] = jnp.dot(u, wf_ref[...], preferred_element_type=jnp.float32)


def _outmlp(parts, w_stack_t, b_stack, wft):
    grid = N // BN
    return pl.pallas_call(
        _mlp_body,
        grid=(grid,),
        in_specs=[pl.BlockSpec((BN, H), lambda b: (b, 0))
                  for _ in range(2 * SL)] + [
            pl.BlockSpec((NUM_OUT_LAYERS, H, H), lambda b: (0, 0, 0)),
            pl.BlockSpec((NUM_OUT_LAYERS, H), lambda b: (0, 0)),
            pl.BlockSpec((H, H), lambda b: (0, 0)),
        ],
        out_specs=pl.BlockSpec((BN, H), lambda b: (b, 0)),
        out_shape=jax.ShapeDtypeStruct((N, H), jnp.float32),
    )(*parts, w_stack_t, b_stack, wft)


def kernel(x, pos, edge_index, freq, W_x, W_rbf_emb, b_rbf_emb, W_emb, b_emb,
           W_rbf_out, W_out_lins, b_out_lins, W_final):
    Wi = W_emb[:, :H]
    Wj = W_emb[:, H:2 * H]
    Wr = W_emb[:, 2 * H:]

    ti, tj = _prep(x, W_x.T, Wi.T, Wj.T)

    px = pos[:, 0]
    py = pos[:, 1]
    pz = pos[:, 2]
    idx_j = edge_index[0]
    idx_i = edge_index[1]

    freq_p = jnp.pad(freq, (0, 2)).reshape(1, 8)
    wrbfe_p = jnp.pad(W_rbf_emb.T, ((0, 2), (0, 0)))
    wrbfo_p = jnp.pad(W_rbf_out.T, ((0, 2), (0, 0)))
    e_ar = jnp.arange(BE, dtype=jnp.int32)
    onehot = (e_ar[:, None] // H == jnp.arange(GR, dtype=jnp.int32)[None, :]
              ).astype(jnp.float32)
    sel = (e_ar[:, None] % H == jnp.arange(H, dtype=jnp.int32)[None, :]
           ).astype(jnp.float32)
    zeros_nh = jnp.zeros((NP, H), jnp.float32)

    parts = []
    for s in range(SL):
        ii_s = lax.slice_in_dim(idx_i, s * ES, (s + 1) * ES)
        jj_s = lax.slice_in_dim(idx_j, s * ES, (s + 1) * ES)
        g, d2 = _gather(ti, tj, px, py, pz, ii_s, jj_s)
        d2r = d2.reshape(EPS // H, H)
        t = _edge(g, d2r, freq_p, onehot, sel, wrbfe_p,
                  b_rbf_emb.reshape(1, H), Wr.T, b_emb.reshape(1, H), wrbfo_p)
        ps = _scatter(t, ii_s, zeros_nh)
        parts.append(ps[0])
        parts.append(ps[1])

    w_stack_t = jnp.transpose(W_out_lins, (0, 2, 1))
    P = _outmlp(parts, w_stack_t, b_out_lins, W_final.T)
    return P


# speedup vs baseline: 6.1412x; 1.1715x over previous
import functools

import jax
import jax.numpy as jnp
from jax import lax
from jax.experimental import pallas as pl
from jax.experimental.pallas import tpu as pltpu
from jax.experimental.pallas import tpu_sc as plsc

N = 10000
E = 320000
H = 128
CUTOFF = 5.0
P_EXP = 5
NUM_OUT_LAYERS = 3

NC = 2
NS = 16
NW = NC * NS
CHUNK = 128
SL = 4
ES = E // SL
NCHUNK = ES // CHUNK
CPW = -(-NCHUNK // NW)
NP = 10240
ROWS_PER_SUB = NP // NS
EPS = 81920

BN = 2000
BE = 4096


def _swish(v):
    return v * (1.0 / (1.0 + jnp.exp(-v)))


def _prep_body(x_ref, wxt_ref, wit_ref, wjt_ref, ti_ref, tj_ref):
    h = jnp.dot(x_ref[...], wxt_ref[...], preferred_element_type=jnp.float32)
    ti_ref[...] = jnp.dot(h, wit_ref[...], preferred_element_type=jnp.float32)
    tj_ref[...] = jnp.dot(h, wjt_ref[...], preferred_element_type=jnp.float32)


def _prep(x, wxt, wit, wjt):
    grid = N // BN
    return pl.pallas_call(
        _prep_body,
        grid=(grid,),
        in_specs=[
            pl.BlockSpec((BN, H), lambda b: (b, 0)),
            pl.BlockSpec((H, H), lambda b: (0, 0)),
            pl.BlockSpec((H, H), lambda b: (0, 0)),
            pl.BlockSpec((H, H), lambda b: (0, 0)),
        ],
        out_specs=[
            pl.BlockSpec((BN, H), lambda b: (b, 0)),
            pl.BlockSpec((BN, H), lambda b: (b, 0)),
        ],
        out_shape=[
            jax.ShapeDtypeStruct((N, H), jnp.float32),
            jax.ShapeDtypeStruct((N, H), jnp.float32),
        ],
    )(x, wxt, wit, wjt)


def _gather_body(ti_hbm, tj_hbm, px_hbm, py_hbm, pz_hbm, ii_hbm, jj_hbm,
                 g_hbm, d2_hbm,
                 px_v, py_v, pz_v,
                 ii0, ii1, jj0, jj1, gi0, gi1, gj0, gj1, d20, d21,
                 si0, si1, sj0, sj1):
    wid = lax.axis_index("s") * NC + lax.axis_index("c")
    pltpu.sync_copy(px_hbm, px_v)
    pltpu.sync_copy(py_hbm, py_v)
    pltpu.sync_copy(pz_hbm, pz_v)
    ii = (ii0, ii1)
    jj = (jj0, jj1)
    gi = (gi0, gi1)
    gj = (gj0, gj1)
    d2 = (d20, d21)
    si = (si0, si1)
    sj = (sj0, sj1)

    def fire(k, b):
        @pl.when(wid + NW * k < NCHUNK)
        def _():
            base = pl.multiple_of((wid + NW * k) * CHUNK, CHUNK)
            pltpu.sync_copy(ii_hbm.at[pl.ds(base, CHUNK)], ii[b])
            pltpu.sync_copy(jj_hbm.at[pl.ds(base, CHUNK)], jj[b])
            pltpu.async_copy(ti_hbm.at[ii[b]], gi[b], si[b])
            pltpu.async_copy(tj_hbm.at[jj[b]], gj[b], sj[b])

    def process(k, b):
        @pl.when(wid + NW * k < NCHUNK)
        def _():
            base = pl.multiple_of((wid + NW * k) * CHUNK, CHUNK)
            for q in range(CHUNK // 16):
                sl = pl.ds(q * 16, 16)
                iq = ii[b][sl]
                jq = jj[b][sl]
                dx = plsc.load_gather(px_v, [iq]) - plsc.load_gather(px_v, [jq])
                dy = plsc.load_gather(py_v, [iq]) - plsc.load_gather(py_v, [jq])
                dz = plsc.load_gather(pz_v, [iq]) - plsc.load_gather(pz_v, [jq])
                d2[b][sl] = dx * dx + dy * dy + dz * dz
            pltpu.make_async_copy(ti_hbm.at[pl.ds(0, CHUNK)], gi[b],
                                  si[b]).wait()
            pltpu.make_async_copy(tj_hbm.at[pl.ds(0, CHUNK)], gj[b],
                                  sj[b]).wait()

            def e_body(e, carry2):
                for v in range(H // 16):
                    sl = pl.ds(v * 16, 16)
                    gi[b][e, sl] = gi[b][e, sl] + gj[b][e, sl]
                return carry2

            lax.fori_loop(0, CHUNK, e_body, 0)
            pltpu.sync_copy(gi[b], g_hbm.at[pl.ds(base, CHUNK)])
            pltpu.sync_copy(d2[b], d2_hbm.at[pl.ds(base, CHUNK)])

    fire(0, 0)

    def body2(i, carry):
        k = i * 2
        fire(k + 1, 1)
        process(k, 0)
        fire(k + 2, 0)
        process(k + 1, 1)
        return carry

    lax.fori_loop(0, CPW // 2, body2, 0)


def _gather(ti, tj, px, py, pz, idx_i, idx_j):
    mesh = plsc.VectorSubcoreMesh(core_axis_name="c", subcore_axis_name="s", num_cores=NC, num_subcores=NS)
    f = pl.kernel(
        _gather_body,
        out_type=[
            jax.ShapeDtypeStruct((EPS, H), jnp.float32),
            jax.ShapeDtypeStruct((EPS,), jnp.float32),
        ],
        mesh=mesh,
        scratch_types=[
            pltpu.VMEM((N,), jnp.float32),
            pltpu.VMEM((N,), jnp.float32),
            pltpu.VMEM((N,), jnp.float32),
            pltpu.VMEM((CHUNK,), jnp.int32),
            pltpu.VMEM((CHUNK,), jnp.int32),
            pltpu.VMEM((CHUNK,), jnp.int32),
            pltpu.VMEM((CHUNK,), jnp.int32),
            pltpu.VMEM((CHUNK, H), jnp.float32),
            pltpu.VMEM((CHUNK, H), jnp.float32),
            pltpu.VMEM((CHUNK, H), jnp.float32),
            pltpu.VMEM((CHUNK, H), jnp.float32),
            pltpu.VMEM((CHUNK,), jnp.float32),
            pltpu.VMEM((CHUNK,), jnp.float32),
            pltpu.SemaphoreType.DMA,
            pltpu.SemaphoreType.DMA,
            pltpu.SemaphoreType.DMA,
            pltpu.SemaphoreType.DMA,
        ],
        compiler_params=pltpu.CompilerParams(needs_layout_passes=False),
    )
    return f(ti, tj, px, py, pz, idx_i, idx_j)


GR = BE // H


def _edge_body(g_ref, d2r_ref, freq_ref, onehot_ref, sel_ref, wrbfe_ref,
               brbfe_ref, wrt_ref, bemb_ref, wrbfo_ref, t_ref):
    d2b = d2r_ref[...]
    up = jnp.sqrt(d2b) * (1.0 / CUTOFF)
    p = P_EXP
    a = -(p + 1) * (p + 2) / 2.0
    b = p * (p + 2)
    c = -p * (p + 1) / 2.0
    u2 = up * up
    u4 = u2 * u2
    u5 = u4 * up
    u6 = u5 * up
    u7 = u6 * up
    envp = 1.0 / up + a * u5 + b * u6 + c * u7
    envs = jnp.where(d2b > 0, envp, 0.0)
    comps = [envs * jnp.sin(up * freq_ref[0:1, k:k + 1]) for k in range(6)]

    onehot = onehot_ref[...]
    sel = sel_ref[...]

    def unpack(packed):
        m_bcast = jnp.dot(onehot, packed, preferred_element_type=jnp.float32)
        return jnp.sum(m_bcast * sel, axis=1, keepdims=True)

    d2 = unpack(d2b)
    nanmask = jnp.where(d2 == 0, jnp.float32(jnp.nan), jnp.float32(1.0))
    cols = [unpack(cmp) for cmp in comps]
    cols.append(jnp.zeros((BE, 2), jnp.float32))
    rbf = jnp.concatenate(cols, axis=1)
    rbf_h = _swish(
        jnp.dot(rbf, wrbfe_ref[...], preferred_element_type=jnp.float32)
        + brbfe_ref[...])
    rproj = jnp.dot(rbf_h, wrt_ref[...], preferred_element_type=jnp.float32)
    m = _swish(g_ref[...] + rproj + bemb_ref[...])
    t_ref[...] = jnp.dot(
        rbf, wrbfo_ref[...], preferred_element_type=jnp.float32) * m * nanmask


def _edge(g, d2r, freq_p, onehot, sel, wrbfe_p, brbfe, wrt, bemb, wrbfo_p):
    grid = EPS // BE
    return pl.pallas_call(
        _edge_body,
        grid=(grid,),
        in_specs=[
            pl.BlockSpec((BE, H), lambda b: (b, 0)),
            pl.BlockSpec((GR, H), lambda b: (b, 0)),
            pl.BlockSpec((1, 8), lambda b: (0, 0)),
            pl.BlockSpec((BE, GR), lambda b: (0, 0)),
            pl.BlockSpec((BE, H), lambda b: (0, 0)),
            pl.BlockSpec((8, H), lambda b: (0, 0)),
            pl.BlockSpec((1, H), lambda b: (0, 0)),
            pl.BlockSpec((H, H), lambda b: (0, 0)),
            pl.BlockSpec((1, H), lambda b: (0, 0)),
            pl.BlockSpec((8, H), lambda b: (0, 0)),
        ],
        out_specs=pl.BlockSpec((BE, H), lambda b: (b, 0)),
        out_shape=jax.ShapeDtypeStruct((EPS, H), jnp.float32),
    )(g, d2r, freq_p, onehot, sel, wrbfe_p, brbfe, wrt, bemb, wrbfo_p)


def _scatter_body(t_hbm, ii_hbm, zeros_hbm, part_hbm, idx_v, t_v, acc_sh, sem):
    cc = lax.axis_index("c")
    ss = lax.axis_index("s")
    wid = ss * NC + cc

    row0 = pl.multiple_of(ss * ROWS_PER_SUB, ROWS_PER_SUB)
    pltpu.sync_copy(zeros_hbm.at[pl.ds(row0, ROWS_PER_SUB)],
                    acc_sh.at[pl.ds(row0, ROWS_PER_SUB)])
    plsc.subcore_barrier()

    def chunk_body(k, carry):
        cid = wid + NW * k

        @pl.when(cid < NCHUNK)
        def _():
            base = pl.multiple_of(cid * CHUNK, CHUNK)
            pltpu.sync_copy(ii_hbm.at[pl.ds(base, CHUNK)], idx_v)
            pltpu.async_copy(t_hbm.at[pl.ds(base, CHUNK)], t_v, sem).wait()
            pltpu.sync_copy(t_v, acc_sh.at[idx_v], add=True)

        return carry

    lax.fori_loop(0, CPW, chunk_body, 0)
    plsc.subcore_barrier()
    pltpu.sync_copy(acc_sh.at[pl.ds(row0, ROWS_PER_SUB)],
                    part_hbm.at[cc, pl.ds(row0, ROWS_PER_SUB)])


def _scatter(t, idx_i, zeros_nh):
    mesh = plsc.VectorSubcoreMesh(core_axis_name="c", subcore_axis_name="s", num_cores=NC, num_subcores=NS)
    f = pl.kernel(
        _scatter_body,
        out_type=jax.ShapeDtypeStruct((NC, NP, H), jnp.float32),
        mesh=mesh,
        scratch_types=[
            pltpu.VMEM((CHUNK,), jnp.int32),
            pltpu.VMEM((CHUNK, H), jnp.float32),
            pltpu.VMEM_SHARED((NP, H), jnp.float32),
            pltpu.SemaphoreType.DMA,
        ],
        compiler_params=pltpu.CompilerParams(needs_layout_passes=False),
    )
    return f(t, idx_i, zeros_nh)


def _mlp_body(*refs):
    part_refs = refs[:2 * SL]
    w_ref, b_ref, wf_ref, out_ref = refs[2 * SL:]
    u = part_refs[0][...]
    for pr in part_refs[1:]:
        u = u + pr[...]
    for k in range(NUM_OUT_LAYERS):
        u = _swish(
            jnp.dot(u, w_ref[k], preferred_element_type=jnp.float32)
            + b_ref[pl.ds(k, 1)])
    out_ref[...] = jnp.dot(u, wf_ref[...], preferred_element_type=jnp.float32)


def _outmlp(parts, w_stack_t, b_stack, wft):
    grid = N // BN
    return pl.pallas_call(
        _mlp_body,
        grid=(grid,),
        in_specs=[pl.BlockSpec((BN, H), lambda b: (b, 0))
                  for _ in range(2 * SL)] + [
            pl.BlockSpec((NUM_OUT_LAYERS, H, H), lambda b: (0, 0, 0)),
            pl.BlockSpec((NUM_OUT_LAYERS, H), lambda b: (0, 0)),
            pl.BlockSpec((H, H), lambda b: (0, 0)),
        ],
        out_specs=pl.BlockSpec((BN, H), lambda b: (b, 0)),
        out_shape=jax.ShapeDtypeStruct((N, H), jnp.float32),
    )(*parts, w_stack_t, b_stack, wft)


def kernel(x, pos, edge_index, freq, W_x, W_rbf_emb, b_rbf_emb, W_emb, b_emb,
           W_rbf_out, W_out_lins, b_out_lins, W_final):
    Wi = W_emb[:, :H]
    Wj = W_emb[:, H:2 * H]
    Wr = W_emb[:, 2 * H:]

    ti, tj = _prep(x, W_x.T, Wi.T, Wj.T)

    px = pos[:, 0]
    py = pos[:, 1]
    pz = pos[:, 2]
    idx_j = edge_index[0]
    idx_i = edge_index[1]

    freq_p = jnp.pad(freq, (0, 2)).reshape(1, 8)
    wrbfe_p = jnp.pad(W_rbf_emb.T, ((0, 2), (0, 0)))
    wrbfo_p = jnp.pad(W_rbf_out.T, ((0, 2), (0, 0)))
    e_ar = jnp.arange(BE, dtype=jnp.int32)
    onehot = (e_ar[:, None] // H == jnp.arange(GR, dtype=jnp.int32)[None, :]
              ).astype(jnp.float32)
    sel = (e_ar[:, None] % H == jnp.arange(H, dtype=jnp.int32)[None, :]
           ).astype(jnp.float32)
    zeros_nh = jnp.zeros((NP, H), jnp.float32)

    parts = []
    for s in range(SL):
        ii_s = lax.slice_in_dim(idx_i, s * ES, (s + 1) * ES)
        jj_s = lax.slice_in_dim(idx_j, s * ES, (s + 1) * ES)
        g, d2 = _gather(ti, tj, px, py, pz, ii_s, jj_s)
        d2r = d2.reshape(EPS // H, H)
        t = _edge(g, d2r, freq_p, onehot, sel, wrbfe_p,
                  b_rbf_emb.reshape(1, H), Wr.T, b_emb.reshape(1, H), wrbfo_p)
        ps = _scatter(t, ii_s, zeros_nh)
        parts.append(ps[0])
        parts.append(ps[1])

    w_stack_t = jnp.transpose(W_out_lins, (0, 2, 1))
    P = _outmlp(parts, w_stack_t, b_out_lins, W_final.T)
    return P


# speedup vs baseline: 6.7400x; 1.0975x over previous
import functools

import jax
import jax.numpy as jnp
from jax import lax
from jax.experimental import pallas as pl
from jax.experimental.pallas import tpu as pltpu
from jax.experimental.pallas import tpu_sc as plsc

N = 10000
E = 320000
H = 128
CUTOFF = 5.0
P_EXP = 5
NUM_OUT_LAYERS = 3

NC = 2
NS = 16
NW = NC * NS
CHUNK = 128
SL = 4
ES = E // SL
NCHUNK = ES // CHUNK
CPW = -(-NCHUNK // NW)
NP = 10240
ROWS_PER_SUB = NP // NS
EPS = 81920

BN = 2000
BE = 4096


def _swish(v):
    return v * (1.0 / (1.0 + jnp.exp(-v)))


def _prep_body(x_ref, wxt_ref, wit_ref, wjt_ref, ti_ref, tj_ref):
    h = jnp.dot(x_ref[...], wxt_ref[...], preferred_element_type=jnp.float32)
    ti_ref[...] = jnp.dot(h, wit_ref[...], preferred_element_type=jnp.float32)
    tj_ref[...] = jnp.dot(h, wjt_ref[...], preferred_element_type=jnp.float32)


def _prep(x, wxt, wit, wjt):
    grid = N // BN
    return pl.pallas_call(
        _prep_body,
        grid=(grid,),
        in_specs=[
            pl.BlockSpec((BN, H), lambda b: (b, 0)),
            pl.BlockSpec((H, H), lambda b: (0, 0)),
            pl.BlockSpec((H, H), lambda b: (0, 0)),
            pl.BlockSpec((H, H), lambda b: (0, 0)),
        ],
        out_specs=[
            pl.BlockSpec((BN, H), lambda b: (b, 0)),
            pl.BlockSpec((BN, H), lambda b: (b, 0)),
        ],
        out_shape=[
            jax.ShapeDtypeStruct((N, H), jnp.float32),
            jax.ShapeDtypeStruct((N, H), jnp.float32),
        ],
    )(x, wxt, wit, wjt)


def _gather_body(ti_hbm, tj_hbm, px_hbm, py_hbm, pz_hbm, ii_hbm, jj_hbm,
                 g_hbm, d2_hbm,
                 px_v, py_v, pz_v,
                 ii0, ii1, jj0, jj1, gi0, gi1, gj0, gj1, d20, d21,
                 si0, si1, sj0, sj1):
    wid = lax.axis_index("s") * NC + lax.axis_index("c")
    pltpu.sync_copy(px_hbm, px_v)
    pltpu.sync_copy(py_hbm, py_v)
    pltpu.sync_copy(pz_hbm, pz_v)
    ii = (ii0, ii1)
    jj = (jj0, jj1)
    gi = (gi0, gi1)
    gj = (gj0, gj1)
    d2 = (d20, d21)
    si = (si0, si1)
    sj = (sj0, sj1)

    def fire(k, b):
        @pl.when(wid + NW * k < NCHUNK)
        def _():
            base = pl.multiple_of((wid + NW * k) * CHUNK, CHUNK)
            pltpu.sync_copy(ii_hbm.at[pl.ds(base, CHUNK)], ii[b])
            pltpu.sync_copy(jj_hbm.at[pl.ds(base, CHUNK)], jj[b])
            pltpu.async_copy(ti_hbm.at[ii[b]], gi[b], si[b])
            pltpu.async_copy(tj_hbm.at[jj[b]], gj[b], sj[b])

    def process(k, b):
        @pl.when(wid + NW * k < NCHUNK)
        def _():
            base = pl.multiple_of((wid + NW * k) * CHUNK, CHUNK)
            for q in range(CHUNK // 16):
                sl = pl.ds(q * 16, 16)
                iq = ii[b][sl]
                jq = jj[b][sl]
                dx = plsc.load_gather(px_v, [iq]) - plsc.load_gather(px_v, [jq])
                dy = plsc.load_gather(py_v, [iq]) - plsc.load_gather(py_v, [jq])
                dz = plsc.load_gather(pz_v, [iq]) - plsc.load_gather(pz_v, [jq])
                d2[b][sl] = dx * dx + dy * dy + dz * dz
            pltpu.make_async_copy(ti_hbm.at[pl.ds(0, CHUNK)], gi[b],
                                  si[b]).wait()
            pltpu.make_async_copy(tj_hbm.at[pl.ds(0, CHUNK)], gj[b],
                                  sj[b]).wait()

            def e_body(e, carry2):
                for v in range(H // 16):
                    sl = pl.ds(v * 16, 16)
                    gi[b][e, sl] = gi[b][e, sl] + gj[b][e, sl]
                return carry2

            lax.fori_loop(0, CHUNK, e_body, 0)
            pltpu.sync_copy(gi[b], g_hbm.at[pl.ds(base, CHUNK)])
            pltpu.sync_copy(d2[b], d2_hbm.at[pl.ds(base, CHUNK)])

    fire(0, 0)

    def body2(i, carry):
        k = i * 2
        fire(k + 1, 1)
        process(k, 0)
        fire(k + 2, 0)
        process(k + 1, 1)
        return carry

    lax.fori_loop(0, CPW // 2, body2, 0)


def _gather(ti, tj, px, py, pz, idx_i, idx_j):
    mesh = plsc.VectorSubcoreMesh(core_axis_name="c", subcore_axis_name="s", num_cores=NC, num_subcores=NS)
    f = pl.kernel(
        _gather_body,
        out_type=[
            jax.ShapeDtypeStruct((EPS, H), jnp.float32),
            jax.ShapeDtypeStruct((EPS,), jnp.float32),
        ],
        mesh=mesh,
        scratch_types=[
            pltpu.VMEM((N,), jnp.float32),
            pltpu.VMEM((N,), jnp.float32),
            pltpu.VMEM((N,), jnp.float32),
            pltpu.VMEM((CHUNK,), jnp.int32),
            pltpu.VMEM((CHUNK,), jnp.int32),
            pltpu.VMEM((CHUNK,), jnp.int32),
            pltpu.VMEM((CHUNK,), jnp.int32),
            pltpu.VMEM((CHUNK, H), jnp.float32),
            pltpu.VMEM((CHUNK, H), jnp.float32),
            pltpu.VMEM((CHUNK, H), jnp.float32),
            pltpu.VMEM((CHUNK, H), jnp.float32),
            pltpu.VMEM((CHUNK,), jnp.float32),
            pltpu.VMEM((CHUNK,), jnp.float32),
            pltpu.SemaphoreType.DMA,
            pltpu.SemaphoreType.DMA,
            pltpu.SemaphoreType.DMA,
            pltpu.SemaphoreType.DMA,
        ],
        compiler_params=pltpu.CompilerParams(needs_layout_passes=False),
    )
    return f(ti, tj, px, py, pz, idx_i, idx_j)


GR = BE // H


def _edge_body(g_ref, d2r_ref, freq_ref, onehot_ref, sel_ref, wrbfe_ref,
               brbfe_ref, wrt_ref, bemb_ref, wrbfo_ref, t_ref):
    d2b = d2r_ref[...]
    up = jnp.sqrt(d2b) * (1.0 / CUTOFF)
    p = P_EXP
    a = -(p + 1) * (p + 2) / 2.0
    b = p * (p + 2)
    c = -p * (p + 1) / 2.0
    u2 = up * up
    u4 = u2 * u2
    u5 = u4 * up
    u6 = u5 * up
    u7 = u6 * up
    envp = 1.0 / up + a * u5 + b * u6 + c * u7
    envs = jnp.where(d2b > 0, envp, 0.0)
    comps = [envs * jnp.sin(up * freq_ref[0:1, k:k + 1]) for k in range(6)]

    onehot = onehot_ref[...]
    sel = sel_ref[...]

    def unpack(packed):
        m_bcast = jnp.dot(onehot, packed, preferred_element_type=jnp.float32)
        return jnp.sum(m_bcast * sel, axis=1, keepdims=True)

    d2 = unpack(d2b)
    nanmask = jnp.where(d2 == 0, jnp.float32(jnp.nan), jnp.float32(1.0))
    cols = [unpack(cmp) for cmp in comps]
    cols.append(jnp.zeros((BE, 2), jnp.float32))
    rbf = jnp.concatenate(cols, axis=1)
    rbf_h = _swish(
        jnp.dot(rbf, wrbfe_ref[...], preferred_element_type=jnp.float32)
        + brbfe_ref[...])
    rproj = jnp.dot(rbf_h, wrt_ref[...], preferred_element_type=jnp.float32)
    m = _swish(g_ref[...] + rproj + bemb_ref[...])
    t_ref[...] = jnp.dot(
        rbf, wrbfo_ref[...], preferred_element_type=jnp.float32) * m * nanmask


def _edge(g, d2r, freq_p, onehot, sel, wrbfe_p, brbfe, wrt, bemb, wrbfo_p):
    grid = EPS // BE
    return pl.pallas_call(
        _edge_body,
        grid=(grid,),
        in_specs=[
            pl.BlockSpec((BE, H), lambda b: (b, 0)),
            pl.BlockSpec((GR, H), lambda b: (b, 0)),
            pl.BlockSpec((1, 8), lambda b: (0, 0)),
            pl.BlockSpec((BE, GR), lambda b: (0, 0)),
            pl.BlockSpec((BE, H), lambda b: (0, 0)),
            pl.BlockSpec((8, H), lambda b: (0, 0)),
            pl.BlockSpec((1, H), lambda b: (0, 0)),
            pl.BlockSpec((H, H), lambda b: (0, 0)),
            pl.BlockSpec((1, H), lambda b: (0, 0)),
            pl.BlockSpec((8, H), lambda b: (0, 0)),
        ],
        out_specs=pl.BlockSpec((BE, H), lambda b: (b, 0)),
        out_shape=jax.ShapeDtypeStruct((EPS, H), jnp.float32),
    )(g, d2r, freq_p, onehot, sel, wrbfe_p, brbfe, wrt, bemb, wrbfo_p)


def _scatter_body(t_hbm, ii_hbm, zeros_hbm, part_hbm, idx0, idx1, t0, t1,
                  acc_sh, se0, se1):
    cc = lax.axis_index("c")
    ss = lax.axis_index("s")
    wid = ss * NC + cc
    idx = (idx0, idx1)
    tv = (t0, t1)
    se = (se0, se1)

    row0 = pl.multiple_of(ss * ROWS_PER_SUB, ROWS_PER_SUB)
    pltpu.sync_copy(zeros_hbm.at[pl.ds(row0, ROWS_PER_SUB)],
                    acc_sh.at[pl.ds(row0, ROWS_PER_SUB)])
    plsc.subcore_barrier()

    def fire(k, b):
        @pl.when(wid + NW * k < NCHUNK)
        def _():
            base = pl.multiple_of((wid + NW * k) * CHUNK, CHUNK)
            pltpu.sync_copy(ii_hbm.at[pl.ds(base, CHUNK)], idx[b])
            pltpu.async_copy(t_hbm.at[pl.ds(base, CHUNK)], tv[b], se[b])

    def process(k, b):
        @pl.when(wid + NW * k < NCHUNK)
        def _():
            pltpu.make_async_copy(t_hbm.at[pl.ds(0, CHUNK)], tv[b],
                                  se[b]).wait()
            pltpu.sync_copy(tv[b], acc_sh.at[idx[b]], add=True)

    fire(0, 0)

    def body2(i, carry):
        k = i * 2
        fire(k + 1, 1)
        process(k, 0)
        fire(k + 2, 0)
        process(k + 1, 1)
        return carry

    lax.fori_loop(0, CPW // 2, body2, 0)
    plsc.subcore_barrier()
    pltpu.sync_copy(acc_sh.at[pl.ds(row0, ROWS_PER_SUB)],
                    part_hbm.at[cc, pl.ds(row0, ROWS_PER_SUB)])


def _scatter(t, idx_i, zeros_nh):
    mesh = plsc.VectorSubcoreMesh(core_axis_name="c", subcore_axis_name="s", num_cores=NC, num_subcores=NS)
    f = pl.kernel(
        _scatter_body,
        out_type=jax.ShapeDtypeStruct((NC, NP, H), jnp.float32),
        mesh=mesh,
        scratch_types=[
            pltpu.VMEM((CHUNK,), jnp.int32),
            pltpu.VMEM((CHUNK,), jnp.int32),
            pltpu.VMEM((CHUNK, H), jnp.float32),
            pltpu.VMEM((CHUNK, H), jnp.float32),
            pltpu.VMEM_SHARED((NP, H), jnp.float32),
            pltpu.SemaphoreType.DMA,
            pltpu.SemaphoreType.DMA,
        ],
        compiler_params=pltpu.CompilerParams(needs_layout_passes=False),
    )
    return f(t, idx_i, zeros_nh)


def _mlp_body(*refs):
    part_refs = refs[:2 * SL]
    w_ref, b_ref, wf_ref, out_ref = refs[2 * SL:]
    u = part_refs[0][...]
    for pr in part_refs[1:]:
        u = u + pr[...]
    for k in range(NUM_OUT_LAYERS):
        u = _swish(
            jnp.dot(u, w_ref[k], preferred_element_type=jnp.float32)
            + b_ref[pl.ds(k, 1)])
    out_ref[...] = jnp.dot(u, wf_ref[...], preferred_element_type=jnp.float32)


def _outmlp(parts, w_stack_t, b_stack, wft):
    grid = N // BN
    return pl.pallas_call(
        _mlp_body,
        grid=(grid,),
        in_specs=[pl.BlockSpec((BN, H), lambda b: (b, 0))
                  for _ in range(2 * SL)] + [
            pl.BlockSpec((NUM_OUT_LAYERS, H, H), lambda b: (0, 0, 0)),
            pl.BlockSpec((NUM_OUT_LAYERS, H), lambda b: (0, 0)),
            pl.BlockSpec((H, H), lambda b: (0, 0)),
        ],
        out_specs=pl.BlockSpec((BN, H), lambda b: (b, 0)),
        out_shape=jax.ShapeDtypeStruct((N, H), jnp.float32),
    )(*parts, w_stack_t, b_stack, wft)


def kernel(x, pos, edge_index, freq, W_x, W_rbf_emb, b_rbf_emb, W_emb, b_emb,
           W_rbf_out, W_out_lins, b_out_lins, W_final):
    Wi = W_emb[:, :H]
    Wj = W_emb[:, H:2 * H]
    Wr = W_emb[:, 2 * H:]

    ti, tj = _prep(x, W_x.T, Wi.T, Wj.T)

    px = pos[:, 0]
    py = pos[:, 1]
    pz = pos[:, 2]
    idx_j = edge_index[0]
    idx_i = edge_index[1]

    freq_p = jnp.pad(freq, (0, 2)).reshape(1, 8)
    wrbfe_p = jnp.pad(W_rbf_emb.T, ((0, 2), (0, 0)))
    wrbfo_p = jnp.pad(W_rbf_out.T, ((0, 2), (0, 0)))
    e_ar = jnp.arange(BE, dtype=jnp.int32)
    onehot = (e_ar[:, None] // H == jnp.arange(GR, dtype=jnp.int32)[None, :]
              ).astype(jnp.float32)
    sel = (e_ar[:, None] % H == jnp.arange(H, dtype=jnp.int32)[None, :]
           ).astype(jnp.float32)
    zeros_nh = jnp.zeros((NP, H), jnp.float32)

    parts = []
    for s in range(SL):
        ii_s = lax.slice_in_dim(idx_i, s * ES, (s + 1) * ES)
        jj_s = lax.slice_in_dim(idx_j, s * ES, (s + 1) * ES)
        g, d2 = _gather(ti, tj, px, py, pz, ii_s, jj_s)
        d2r = d2.reshape(EPS // H, H)
        t = _edge(g, d2r, freq_p, onehot, sel, wrbfe_p,
                  b_rbf_emb.reshape(1, H), Wr.T, b_emb.reshape(1, H), wrbfo_p)
        ps = _scatter(t, ii_s, zeros_nh)
        parts.append(ps[0])
        parts.append(ps[1])

    w_stack_t = jnp.transpose(W_out_lins, (0, 2, 1))
    P = _outmlp(parts, w_stack_t, b_out_lins, W_final.T)
    return P
